# baseline (speedup 1.0000x reference)
import functools
import math

import jax
import jax.numpy as jnp
from jax import lax
from jax.experimental import pallas as pl
from jax.experimental.pallas import tpu as pltpu
from jax.experimental.pallas import tpu_sc as plsc

_LOAD_OFFS = (0, 16, 32, 48, 64, 80, 84)


def _mask_body(freq_ref, out_ref, *, n_drop, f):
    fblk = freq_ref[...][:, :f]
    r = fblk.shape[0]
    idx = lax.broadcasted_iota(jnp.int32, (r, f), 1)
    cur = fblk
    drop = jnp.zeros((r, f), jnp.bool_)
    for _ in range(n_drop):
        m = jnp.min(cur, axis=1, keepdims=True)
        ism = cur == m
        pos = jnp.max(jnp.where(ism, idx, -1), axis=1, keepdims=True)
        hit = idx == pos
        drop = jnp.logical_or(drop, hit)
        cur = jnp.where(hit, jnp.float32(jnp.inf), cur)
    mask = jnp.where(drop, jnp.float32(0.0), jnp.float32(1.0))
    ones = jnp.ones((r, 1), jnp.float32)
    out_ref[...] = jnp.concatenate([ones, mask], axis=1)


def _feature_mask(freq, f, n_drop, block_rows=512):
    b, fw = freq.shape
    return pl.pallas_call(
        functools.partial(_mask_body, n_drop=n_drop, f=f),
        grid=(b // block_rows,),
        in_specs=[pl.BlockSpec((block_rows, fw), lambda i: (i, 0))],
        out_specs=pl.BlockSpec((block_rows, f + 1), lambda i: (i, 0)),
        out_shape=jax.ShapeDtypeStruct((b, f + 1), jnp.float32),
    )(freq)


def _sc_gather(table, ids):
    n, f = table.shape
    (b,) = ids.shape
    info = plsc.get_sparse_core_info()
    nc, ns, nl = info.num_cores, info.num_subcores, info.num_lanes
    nw = nc * ns
    b_per_w = b // nw
    chunk = 128
    kc = b_per_w // chunk
    f2 = 2 * f

    table2 = table.reshape(n // 2, f2)
    hids = ids // 2
    par = ids % 2

    mesh = plsc.VectorSubcoreMesh(core_axis_name="c", subcore_axis_name="s")

    @functools.partial(
        pl.kernel,
        mesh=mesh,
        compiler_params=pltpu.CompilerParams(use_tc_tiling_on_sc=False),
        out_type=jax.ShapeDtypeStruct((b, 128), jnp.float32),
        scratch_types=[
            pltpu.VMEM((chunk,), jnp.int32),
            pltpu.VMEM((chunk,), jnp.int32),
            pltpu.VMEM((chunk, f2), jnp.float32),
            pltpu.VMEM((chunk, 128), jnp.float32),
            pltpu.SemaphoreType.DMA,
        ],
    )
    def gather_rows(table_hbm, hids_hbm, par_hbm, out_hbm, hidx_v, par_v,
                    win_v, rows_v, sem):
        wid = lax.axis_index("s") * nc + lax.axis_index("c")
        base = wid * b_per_w
        for c in range(kc):
            lo = base + c * chunk
            pltpu.sync_copy(hids_hbm.at[pl.ds(lo, chunk)], hidx_v)
            pltpu.sync_copy(par_hbm.at[pl.ds(lo, chunk)], par_v)
            pltpu.async_copy(table_hbm.at[hidx_v], win_v, sem).wait()

            def extract(blk, carry):
                vecp = par_v[pl.ds(blk * nl, nl)]
                for k in range(nl):
                    s = blk * nl + k
                    p = vecp[k]

                    @pl.when(p == 0)
                    def _():
                        for o in _LOAD_OFFS:
                            rows_v[s, pl.ds(o, nl)] = win_v[s, pl.ds(o, nl)]

                    @pl.when(p != 0)
                    def _():
                        for o in _LOAD_OFFS:
                            rows_v[s, pl.ds(o, nl)] = (
                                win_v[s, pl.ds(f + o, nl)]
                            )

                return carry

            lax.fori_loop(0, chunk // nl, extract, 0)
            pltpu.sync_copy(rows_v, out_hbm.at[pl.ds(lo, chunk)])

    return gather_rows(table2, hids, par)


def kernel(x_num, sample_feature_frequency, sample_ids):
    b, f = x_num.shape
    n_remain = min(math.ceil(f * (1.0 - 0.15)), f - 1)
    n_drop = f - n_remain

    freq = _sc_gather(sample_feature_frequency, sample_ids)
    mask = _feature_mask(freq, f, n_drop)
    return mask[:, :, None]


# speedup vs baseline: 4.3762x; 4.3762x over previous
import functools
import math

import jax
import jax.numpy as jnp
from jax import lax
from jax.experimental import pallas as pl
from jax.experimental.pallas import tpu as pltpu
from jax.experimental.pallas import tpu_sc as plsc


def _mask_body(freq_ref, out_ref, *, n_drop):
    fblk = freq_ref[...]
    r, f = fblk.shape
    idx = lax.broadcasted_iota(jnp.int32, (r, f), 1)
    cur = fblk
    drop = jnp.zeros((r, f), jnp.bool_)
    for _ in range(n_drop):
        m = jnp.min(cur, axis=1, keepdims=True)
        ism = cur == m
        pos = jnp.max(jnp.where(ism, idx, -1), axis=1, keepdims=True)
        hit = idx == pos
        drop = jnp.logical_or(drop, hit)
        cur = jnp.where(hit, jnp.float32(jnp.inf), cur)
    mask = jnp.where(drop, jnp.float32(0.0), jnp.float32(1.0))
    ones = jnp.ones((r, 1), jnp.float32)
    out_ref[...] = jnp.concatenate([ones, mask], axis=1)


def _feature_mask(freq, n_drop, block_rows=512):
    b, f = freq.shape
    return pl.pallas_call(
        functools.partial(_mask_body, n_drop=n_drop),
        grid=(b // block_rows,),
        in_specs=[pl.BlockSpec((block_rows, f), lambda i: (i, 0))],
        out_specs=pl.BlockSpec((block_rows, f + 1), lambda i: (i, 0)),
        out_shape=jax.ShapeDtypeStruct((b, f + 1), jnp.float32),
    )(freq)


def _sc_gather(table, ids):
    n, f = table.shape
    (b,) = ids.shape
    info = plsc.get_sparse_core_info()
    nc, ns, nl = info.num_cores, info.num_subcores, info.num_lanes
    nw = nc * ns
    b_per_w = b // nw
    ids2 = ids.reshape(nw, b_per_w)

    mesh = plsc.VectorSubcoreMesh(core_axis_name="c", subcore_axis_name="s")

    @functools.partial(
        pl.kernel,
        mesh=mesh,
        out_type=jax.ShapeDtypeStruct((b, f), jnp.float32),
        scratch_types=[
            pltpu.VMEM((b_per_w,), jnp.int32),
            pltpu.VMEM((b_per_w, f), jnp.float32),
            pltpu.SemaphoreType.DMA,
        ],
    )
    def gather_rows(table_hbm, ids_hbm, out_hbm, idx_v, rows_v, sem):
        wid = lax.axis_index("s") * nc + lax.axis_index("c")
        base = wid * b_per_w
        pltpu.sync_copy(ids_hbm.at[wid], idx_v)

        def issue(blk, carry):
            vec = idx_v[pl.ds(blk * nl, nl)]
            for k in range(nl):
                i = vec[k]
                pltpu.async_copy(table_hbm.at[i], rows_v.at[blk * nl + k], sem)
            return carry

        lax.fori_loop(0, b_per_w // nl, issue, 0)
        pltpu.make_async_copy(out_hbm.at[pl.ds(base, b_per_w)], rows_v, sem).wait()
        pltpu.sync_copy(rows_v, out_hbm.at[pl.ds(base, b_per_w)])

    return gather_rows(table, ids2)


def kernel(x_num, sample_feature_frequency, sample_ids):
    b, f = x_num.shape
    n_remain = min(math.ceil(f * (1.0 - 0.15)), f - 1)
    n_drop = f - n_remain

    freq = _sc_gather(sample_feature_frequency, sample_ids)
    mask = _feature_mask(freq, n_drop)
    return mask[:, :, None]
